# baseline (speedup 1.0000x reference)
import jax
import jax.numpy as jnp
from jax import lax
from jax.experimental import pallas as pl
from jax.experimental.pallas import tpu as pltpu
from jax.experimental.pallas import tpu_sc as plsc

N = 10000
NP = 10240
E = 320000
D = 128
CW = 8
WA = D + CW
PW = 16
NC, NS = 2, 16
NW = NC * NS
EPW = E // NW
CH = 100
NCHUNK = EPW // CH
RPT = NP // NS
FP32 = jnp.float32


def _make_agg(width):
  mesh = plsc.VectorSubcoreMesh(
      core_axis_name="c", subcore_axis_name="s", num_cores=NC, num_subcores=NS)
  scratch = [
      pltpu.VMEM((CH, width), FP32),
      pltpu.VMEM((CH, width), FP32),
      pltpu.VMEM((2, CH), jnp.int32),
      pltpu.VMEM((2, CH), jnp.int32),
      pltpu.VMEM((2, CH), jnp.int32),
      pltpu.VMEM((2, CH), jnp.int32),
      pltpu.VMEM_SHARED((NP, width), FP32),
  ] + [pltpu.SemaphoreType.DMA] * 8
  out_type = jax.ShapeDtypeStruct((NC, NP, width), FP32)

  def body(table, eim, zrows, *refs):
    (sum_out, rows0, rows1, is0, is1, is2, is3,
     acc_sh, i0, i1, i2, i3, g0, g1, ss0, ss1) = refs
    rows = (rows0, rows1)
    islot = (is0, is1, is2, is3)
    isem = (i0, i1, i2, i3)
    gsem = (g0, g1)
    ssem = (ss0, ss1)
    c = lax.axis_index("c")
    s = lax.axis_index("s")
    wid = s * NC + c

    base = s * RPT
    pltpu.sync_copy(zrows, acc_sh.at[pl.ds(base, RPT)])
    plsc.subcore_barrier()

    for cc in (0, 1):
      pltpu.async_copy(eim.at[wid, cc], islot[cc], isem[cc])

    def do_chunk(cdyn, k):
      b = k % 2
      s2 = (k + 2) % 4
      pltpu.make_async_copy(eim.at[wid, 0], islot[k], isem[k]).wait()
      @pl.when(cdyn >= 2)
      def _():
        pltpu.make_async_copy(rows[b], acc_sh.at[pl.ds(0, CH)], ssem[b]).wait()
      gd = pltpu.async_copy(table.at[islot[k].at[0]], rows[b], gsem[b])
      @pl.when(cdyn + 2 < NCHUNK)
      def _():
        pltpu.async_copy(eim.at[wid, cdyn + 2], islot[s2], isem[s2])
      gd.wait()
      pltpu.async_copy(rows[b], acc_sh.at[islot[k].at[1]], ssem[b], add=True)

    def quad(t, _):
      cb = t * 4
      for k in range(4):
        do_chunk(cb + k, k)
      return 0
    lax.fori_loop(0, NCHUNK // 4, quad, 0)
    for b in (0, 1):
      pltpu.make_async_copy(rows[b], acc_sh.at[pl.ds(0, CH)], ssem[b]).wait()
    plsc.subcore_barrier()

    pltpu.sync_copy(acc_sh.at[pl.ds(base, RPT)], sum_out.at[c, pl.ds(base, RPT)])

  return pl.kernel(
      body, out_type=out_type, mesh=mesh, scratch_types=scratch,
      compiler_params=pltpu.CompilerParams(use_tc_tiling_on_sc=False))


_agg_l1 = _make_agg(WA)
_agg_l2 = _make_agg(PW)

_TCR = 1000


def _tc1_body(acc_ref, x_ref, w1l_ref, b1_ref, w1r_ref,
              w2l_ref, w2r_ref, p_ref, q_ref, cnt_ref):
  a0 = acc_ref[0]
  a1 = acc_ref[1]
  cnt = jnp.maximum(a0[:, D:D + 1] + a1[:, D:D + 1], 1.0)
  mean = (a0[:, :D] + a1[:, :D]) / cnt
  h = (jnp.dot(mean, w1l_ref[...], preferred_element_type=FP32)
       + b1_ref[...]
       + jnp.dot(x_ref[...], w1r_ref[...], preferred_element_type=FP32))
  h = jnp.maximum(h, 0.0)
  p_ref[...] = jnp.dot(h, w2l_ref[...], preferred_element_type=FP32)
  q_ref[...] = jnp.dot(h, w2r_ref[...], preferred_element_type=FP32)
  cnt_ref[...] = jnp.broadcast_to(cnt, (cnt.shape[0], CW))


def _tc1(acc, x, w1lt, b1, w1rt, w2lt, w2rt):
  grid = (N // _TCR,)
  return pl.pallas_call(
      _tc1_body,
      grid=grid,
      in_specs=[
          pl.BlockSpec((NC, _TCR, WA), lambda i: (0, i, 0)),
          pl.BlockSpec((_TCR, D), lambda i: (i, 0)),
          pl.BlockSpec((D, D), lambda i: (0, 0)),
          pl.BlockSpec((1, D), lambda i: (0, 0)),
          pl.BlockSpec((D, D), lambda i: (0, 0)),
          pl.BlockSpec((D, PW), lambda i: (0, 0)),
          pl.BlockSpec((D, PW), lambda i: (0, 0)),
      ],
      out_specs=[
          pl.BlockSpec((_TCR, PW), lambda i: (i, 0)),
          pl.BlockSpec((_TCR, PW), lambda i: (i, 0)),
          pl.BlockSpec((_TCR, CW), lambda i: (i, 0)),
      ],
      out_shape=[
          jax.ShapeDtypeStruct((N, PW), FP32),
          jax.ShapeDtypeStruct((N, PW), FP32),
          jax.ShapeDtypeStruct((N, CW), FP32),
      ],
  )(acc, x, w1lt, b1, w1rt, w2lt, w2rt)


def _tc2_body(sump_ref, cnt_ref, q_ref, b2_ref, out_ref):
  cnt = cnt_ref[:, 0:1]
  t = (sump_ref[0] + sump_ref[1]) / cnt + q_ref[...] + b2_ref[...]
  col = lax.broadcasted_iota(jnp.int32, t.shape, 1)
  valid = col < 2
  tm = jnp.where(valid, t, -jnp.inf)
  m = jnp.max(tm, axis=1, keepdims=True)
  ssum = jnp.sum(jnp.where(valid, jnp.exp(t - m), 0.0), axis=1, keepdims=True)
  out_ref[...] = t - m - jnp.log(ssum)


def _tc2(sump, cnt, q, b2):
  grid = (N // _TCR,)
  return pl.pallas_call(
      _tc2_body,
      grid=grid,
      in_specs=[
          pl.BlockSpec((NC, _TCR, PW), lambda i: (0, i, 0)),
          pl.BlockSpec((_TCR, CW), lambda i: (i, 0)),
          pl.BlockSpec((_TCR, PW), lambda i: (i, 0)),
          pl.BlockSpec((1, PW), lambda i: (0, 0)),
      ],
      out_specs=pl.BlockSpec((_TCR, PW), lambda i: (i, 0)),
      out_shape=jax.ShapeDtypeStruct((N, PW), FP32),
  )(sump, cnt, q, b2)


def kernel(x, edge_index, W1l, b1l, W1r, W2l, b2l, W2r):
  e2 = edge_index.astype(jnp.int32).reshape(2, NW, EPW)
  eim = e2.reshape(2, NW, NCHUNK, CH).transpose(1, 2, 0, 3)
  x136 = jnp.pad(x, ((0, 0), (0, CW)), constant_values=1.0)
  zrows_a = jnp.zeros((RPT, WA), FP32)
  zrows_p = jnp.zeros((RPT, PW), FP32)
  sums = _agg_l1(x136, eim, zrows_a)
  w1lt = W1l.T
  w1rt = W1r.T
  w2lt = jnp.zeros((D, PW), FP32).at[:, :2].set(W2l.T)
  w2rt = jnp.zeros((D, PW), FP32).at[:, :2].set(W2r.T)
  b1 = b1l.reshape(1, D)
  b2 = jnp.zeros((1, PW), FP32).at[0, :2].set(b2l)
  p, q, cnt8 = _tc1(sums, x, w1lt, b1, w1rt, w2lt, w2rt)
  sump = _agg_l2(p, eim, zrows_p)
  outp = _tc2(sump, cnt8, q, b2)
  return outp[:, :2]


# speedup vs baseline: 1.0496x; 1.0496x over previous
import jax
import jax.numpy as jnp
from jax import lax
from jax.experimental import pallas as pl
from jax.experimental.pallas import tpu as pltpu
from jax.experimental.pallas import tpu_sc as plsc

N = 10000
NP = 10240
E = 320000
D = 128
CW = 8
WA = D + CW
PW = 16
NC, NS = 2, 16
NW = NC * NS
EPW = E // NW
CH = 100
NCHUNK = EPW // CH
RPT = NP // NS
FP32 = jnp.float32


def _make_agg(width, split_counts):
  mesh = plsc.VectorSubcoreMesh(
      core_axis_name="c", subcore_axis_name="s", num_cores=NC, num_subcores=NS)
  scratch = [
      pltpu.VMEM((CH, width), FP32),
      pltpu.VMEM((CH, width), FP32),
      pltpu.VMEM((2, CH), jnp.int32),
      pltpu.VMEM((2, CH), jnp.int32),
      pltpu.VMEM((2, CH), jnp.int32),
      pltpu.VMEM((2, CH), jnp.int32),
      pltpu.VMEM_SHARED((NP, width), FP32),
  ] + [pltpu.SemaphoreType.DMA] * 8
  if split_counts:
    out_type = (jax.ShapeDtypeStruct((NC, NP, width - CW), FP32),
                jax.ShapeDtypeStruct((NC, NP, CW), FP32))
  else:
    out_type = jax.ShapeDtypeStruct((NC, NP, width), FP32)

  def body(table, eim, zrows, *refs):
    if split_counts:
      (sum_out, cnt_out, rows0, rows1, is0, is1, is2, is3,
       acc_sh, i0, i1, i2, i3, g0, g1, ss0, ss1) = refs
    else:
      (sum_out, rows0, rows1, is0, is1, is2, is3,
       acc_sh, i0, i1, i2, i3, g0, g1, ss0, ss1) = refs
      cnt_out = None
    rows = (rows0, rows1)
    islot = (is0, is1, is2, is3)
    isem = (i0, i1, i2, i3)
    gsem = (g0, g1)
    ssem = (ss0, ss1)
    c = lax.axis_index("c")
    s = lax.axis_index("s")
    wid = s * NC + c

    base = s * RPT
    pltpu.sync_copy(zrows, acc_sh.at[pl.ds(base, RPT)])
    plsc.subcore_barrier()

    for cc in (0, 1):
      pltpu.async_copy(eim.at[wid, cc], islot[cc], isem[cc])

    def do_chunk(cdyn, k):
      b = k % 2
      s2 = (k + 2) % 4
      pltpu.make_async_copy(eim.at[wid, 0], islot[k], isem[k]).wait()
      @pl.when(cdyn >= 2)
      def _():
        pltpu.make_async_copy(rows[b], acc_sh.at[pl.ds(0, CH)], ssem[b]).wait()
      gd = pltpu.async_copy(table.at[islot[k].at[0]], rows[b], gsem[b])
      @pl.when(cdyn + 2 < NCHUNK)
      def _():
        pltpu.async_copy(eim.at[wid, cdyn + 2], islot[s2], isem[s2])
      gd.wait()
      pltpu.async_copy(rows[b], acc_sh.at[islot[k].at[1]], ssem[b], add=True)

    def quad(t, _):
      cb = t * 4
      for k in range(4):
        do_chunk(cb + k, k)
      return 0
    lax.fori_loop(0, NCHUNK // 4, quad, 0)
    for b in (0, 1):
      pltpu.make_async_copy(rows[b], acc_sh.at[pl.ds(0, CH)], ssem[b]).wait()
    plsc.subcore_barrier()

    if split_counts:
      pltpu.sync_copy(acc_sh.at[pl.ds(base, RPT), pl.ds(0, D)],
                      sum_out.at[c, pl.ds(base, RPT)])
      pltpu.sync_copy(acc_sh.at[pl.ds(base, RPT), pl.ds(D, CW)],
                      cnt_out.at[c, pl.ds(base, RPT)])
    else:
      pltpu.sync_copy(acc_sh.at[pl.ds(base, RPT)],
                      sum_out.at[c, pl.ds(base, RPT)])

  return pl.kernel(
      body, out_type=out_type, mesh=mesh, scratch_types=scratch,
      compiler_params=pltpu.CompilerParams(use_tc_tiling_on_sc=False))


_agg_l1 = _make_agg(WA, split_counts=True)
_agg_l2 = _make_agg(PW, split_counts=False)

_TCR = 1000


def _tc1_body(acc_ref, cnt_ref, x_ref, w1l_ref, b1_ref, w1r_ref,
              w2l_ref, w2r_ref, p_ref, q_ref):
  cnt = jnp.maximum(cnt_ref[0][:, 0:1] + cnt_ref[1][:, 0:1], 1.0)
  mean = (acc_ref[0] + acc_ref[1]) / cnt
  h = (jnp.dot(mean, w1l_ref[...], preferred_element_type=FP32)
       + b1_ref[...]
       + jnp.dot(x_ref[...], w1r_ref[...], preferred_element_type=FP32))
  h = jnp.maximum(h, 0.0)
  p_ref[...] = jnp.dot(h, w2l_ref[...], preferred_element_type=FP32)
  q_ref[...] = jnp.dot(h, w2r_ref[...], preferred_element_type=FP32)


def _tc1(acc, cnt, x, w1lt, b1, w1rt, w2lt, w2rt):
  grid = (N // _TCR,)
  return pl.pallas_call(
      _tc1_body,
      grid=grid,
      in_specs=[
          pl.BlockSpec((NC, _TCR, D), lambda i: (0, i, 0)),
          pl.BlockSpec((NC, _TCR, CW), lambda i: (0, i, 0)),
          pl.BlockSpec((_TCR, D), lambda i: (i, 0)),
          pl.BlockSpec((D, D), lambda i: (0, 0)),
          pl.BlockSpec((1, D), lambda i: (0, 0)),
          pl.BlockSpec((D, D), lambda i: (0, 0)),
          pl.BlockSpec((D, PW), lambda i: (0, 0)),
          pl.BlockSpec((D, PW), lambda i: (0, 0)),
      ],
      out_specs=[
          pl.BlockSpec((_TCR, PW), lambda i: (i, 0)),
          pl.BlockSpec((_TCR, PW), lambda i: (i, 0)),
      ],
      out_shape=[
          jax.ShapeDtypeStruct((N, PW), FP32),
          jax.ShapeDtypeStruct((N, PW), FP32),
      ],
  )(acc, cnt, x, w1lt, b1, w1rt, w2lt, w2rt)


def _tc2_body(sump_ref, cnt_ref, q_ref, b2_ref, out_ref):
  cnt = jnp.maximum(cnt_ref[0][:, 0:1] + cnt_ref[1][:, 0:1], 1.0)
  t = (sump_ref[0] + sump_ref[1]) / cnt + q_ref[...] + b2_ref[...]
  col = lax.broadcasted_iota(jnp.int32, t.shape, 1)
  valid = col < 2
  tm = jnp.where(valid, t, -jnp.inf)
  m = jnp.max(tm, axis=1, keepdims=True)
  ssum = jnp.sum(jnp.where(valid, jnp.exp(t - m), 0.0), axis=1, keepdims=True)
  out_ref[...] = t - m - jnp.log(ssum)


def _tc2(sump, cnt, q, b2):
  grid = (N // _TCR,)
  return pl.pallas_call(
      _tc2_body,
      grid=grid,
      in_specs=[
          pl.BlockSpec((NC, _TCR, PW), lambda i: (0, i, 0)),
          pl.BlockSpec((NC, _TCR, CW), lambda i: (0, i, 0)),
          pl.BlockSpec((_TCR, PW), lambda i: (i, 0)),
          pl.BlockSpec((1, PW), lambda i: (0, 0)),
      ],
      out_specs=pl.BlockSpec((_TCR, PW), lambda i: (i, 0)),
      out_shape=jax.ShapeDtypeStruct((N, PW), FP32),
  )(sump, cnt, q, b2)


def kernel(x, edge_index, W1l, b1l, W1r, W2l, b2l, W2r):
  e2 = edge_index.astype(jnp.int32).reshape(2, NW, EPW)
  eim = e2.reshape(2, NW, NCHUNK, CH).transpose(1, 2, 0, 3)
  x136 = jnp.pad(x, ((0, 0), (0, CW)), constant_values=1.0)
  zrows_a = jnp.zeros((RPT, WA), FP32)
  zrows_p = jnp.zeros((RPT, PW), FP32)
  sums, cnts = _agg_l1(x136, eim, zrows_a)
  w1lt = W1l.T
  w1rt = W1r.T
  w2lt = jnp.zeros((D, PW), FP32).at[:, :2].set(W2l.T)
  w2rt = jnp.zeros((D, PW), FP32).at[:, :2].set(W2r.T)
  b1 = b1l.reshape(1, D)
  b2 = jnp.zeros((1, PW), FP32).at[0, :2].set(b2l)
  p, q = _tc1(sums, cnts, x, w1lt, b1, w1rt, w2lt, w2rt)
  sump = _agg_l2(p, eim, zrows_p)
  outp = _tc2(sump, cnts, q, b2)
  return outp[:, :2]


# speedup vs baseline: 1.1426x; 1.0886x over previous
import jax
import jax.numpy as jnp
from jax import lax
from jax.experimental import pallas as pl
from jax.experimental.pallas import tpu as pltpu
from jax.experimental.pallas import tpu_sc as plsc

N = 10000
NP = 10240
E = 320000
D = 128
PW = 16
NC, NS = 2, 16
NW = NC * NS
EPW = E // NW
CH = 100
NCHUNK = EPW // CH
CW = 8
RPT = NP // NS
FP32 = jnp.float32


def _make_agg(width, with_counts):
  mesh = plsc.VectorSubcoreMesh(
      core_axis_name="c", subcore_axis_name="s", num_cores=NC, num_subcores=NS)
  scratch = [
      pltpu.VMEM((CH, width), FP32),
      pltpu.VMEM((CH, width), FP32),
      pltpu.VMEM((2, CH), jnp.int32),
      pltpu.VMEM((2, CH), jnp.int32),
      pltpu.VMEM((2, CH), jnp.int32),
      pltpu.VMEM((2, CH), jnp.int32),
      pltpu.VMEM_SHARED((NP, width), FP32),
  ] + [pltpu.SemaphoreType.DMA] * 8
  out_types = [jax.ShapeDtypeStruct((NC, NP, width), FP32)]
  if with_counts:
    scratch += [
        pltpu.VMEM((CH, CW), FP32),
        pltpu.VMEM_SHARED((NP, CW), FP32),
    ]
    out_types.append(jax.ShapeDtypeStruct((NC, NP, CW), FP32))

  def body(table, eim, zrows, *refs):
    if with_counts:
      (zcnt, ones_h, sum_out, cnt_out, rows0, rows1, is0, is1, is2, is3,
       acc_sh, i0, i1, i2, i3, g0, g1, ss0, ss1, ones_v, cnt_sh) = refs
    else:
      (sum_out, rows0, rows1, is0, is1, is2, is3,
       acc_sh, i0, i1, i2, i3, g0, g1, ss0, ss1) = refs
      zcnt = ones_h = cnt_out = ones_v = cnt_sh = None
    rows = (rows0, rows1)
    islot = (is0, is1, is2, is3)
    isem = (i0, i1, i2, i3)
    gsem = (g0, g1)
    ssem = (ss0, ss1)
    c = lax.axis_index("c")
    s = lax.axis_index("s")
    wid = s * NC + c

    base = s * RPT
    pltpu.sync_copy(zrows, acc_sh.at[pl.ds(base, RPT)])
    if with_counts:
      pltpu.sync_copy(ones_h, ones_v)
      pltpu.sync_copy(zcnt, cnt_sh.at[pl.ds(base, RPT)])
    plsc.subcore_barrier()

    for cc in (0, 1):
      pltpu.async_copy(eim.at[wid, cc], islot[cc], isem[cc])

    def do_chunk(cdyn, k):
      b = k % 2
      s2 = (k + 2) % 4
      pltpu.make_async_copy(eim.at[wid, 0], islot[k], isem[k]).wait()
      @pl.when(cdyn >= 2)
      def _():
        pltpu.make_async_copy(rows[b], acc_sh.at[pl.ds(0, CH)], ssem[b]).wait()
        if with_counts:
          pltpu.make_async_copy(
              ones_v, cnt_sh.at[pl.ds(0, CH)], ssem[b]).wait()
      gd = pltpu.async_copy(table.at[islot[k].at[0]], rows[b], gsem[b])
      @pl.when(cdyn + 2 < NCHUNK)
      def _():
        pltpu.async_copy(eim.at[wid, cdyn + 2], islot[s2], isem[s2])
      gd.wait()
      pltpu.async_copy(rows[b], acc_sh.at[islot[k].at[1]], ssem[b], add=True)
      if with_counts:
        pltpu.async_copy(ones_v, cnt_sh.at[islot[k].at[1]], ssem[b], add=True)

    def quad(t, _):
      cb = t * 4
      for k in range(4):
        do_chunk(cb + k, k)
      return 0
    lax.fori_loop(0, NCHUNK // 4, quad, 0)
    for b in (0, 1):
      pltpu.make_async_copy(rows[b], acc_sh.at[pl.ds(0, CH)], ssem[b]).wait()
      if with_counts:
        pltpu.make_async_copy(ones_v, cnt_sh.at[pl.ds(0, CH)], ssem[b]).wait()
    plsc.subcore_barrier()

    pltpu.sync_copy(acc_sh.at[pl.ds(base, RPT)], sum_out.at[c, pl.ds(base, RPT)])
    if with_counts:
      pltpu.sync_copy(cnt_sh.at[pl.ds(base, RPT)],
                      cnt_out.at[c, pl.ds(base, RPT)])

  out_type = tuple(out_types) if with_counts else out_types[0]
  return pl.kernel(
      body, out_type=out_type, mesh=mesh, scratch_types=scratch,
      compiler_params=pltpu.CompilerParams(use_tc_tiling_on_sc=False))


_agg_l1 = _make_agg(D, with_counts=True)
_agg_l2 = _make_agg(PW, with_counts=False)

_TCR = 1000


def _tc1_body(acc_ref, cnt_ref, x_ref, w1l_ref, b1_ref, w1r_ref,
              w2l_ref, w2r_ref, p_ref, q_ref):
  cnt = jnp.maximum(cnt_ref[0][:, 0:1] + cnt_ref[1][:, 0:1], 1.0)
  mean = (acc_ref[0] + acc_ref[1]) / cnt
  h = (jnp.dot(mean, w1l_ref[...], preferred_element_type=FP32)
       + b1_ref[...]
       + jnp.dot(x_ref[...], w1r_ref[...], preferred_element_type=FP32))
  h = jnp.maximum(h, 0.0)
  p_ref[...] = jnp.dot(h, w2l_ref[...], preferred_element_type=FP32)
  q_ref[...] = jnp.dot(h, w2r_ref[...], preferred_element_type=FP32)


def _tc1(acc, cnt, x, w1lt, b1, w1rt, w2lt, w2rt):
  grid = (N // _TCR,)
  return pl.pallas_call(
      _tc1_body,
      grid=grid,
      in_specs=[
          pl.BlockSpec((NC, _TCR, D), lambda i: (0, i, 0)),
          pl.BlockSpec((NC, _TCR, CW), lambda i: (0, i, 0)),
          pl.BlockSpec((_TCR, D), lambda i: (i, 0)),
          pl.BlockSpec((D, D), lambda i: (0, 0)),
          pl.BlockSpec((1, D), lambda i: (0, 0)),
          pl.BlockSpec((D, D), lambda i: (0, 0)),
          pl.BlockSpec((D, PW), lambda i: (0, 0)),
          pl.BlockSpec((D, PW), lambda i: (0, 0)),
      ],
      out_specs=[
          pl.BlockSpec((_TCR, PW), lambda i: (i, 0)),
          pl.BlockSpec((_TCR, PW), lambda i: (i, 0)),
      ],
      out_shape=[
          jax.ShapeDtypeStruct((N, PW), FP32),
          jax.ShapeDtypeStruct((N, PW), FP32),
      ],
  )(acc, cnt, x, w1lt, b1, w1rt, w2lt, w2rt)


def _tc2_body(sump_ref, cnt_ref, q_ref, b2_ref, out_ref):
  cnt = jnp.maximum(cnt_ref[0][:, 0:1] + cnt_ref[1][:, 0:1], 1.0)
  t = (sump_ref[0] + sump_ref[1]) / cnt + q_ref[...] + b2_ref[...]
  col = lax.broadcasted_iota(jnp.int32, t.shape, 1)
  valid = col < 2
  tm = jnp.where(valid, t, -jnp.inf)
  m = jnp.max(tm, axis=1, keepdims=True)
  ssum = jnp.sum(jnp.where(valid, jnp.exp(t - m), 0.0), axis=1, keepdims=True)
  out_ref[...] = t - m - jnp.log(ssum)


def _tc2(sump, cnt, q, b2):
  grid = (N // _TCR,)
  return pl.pallas_call(
      _tc2_body,
      grid=grid,
      in_specs=[
          pl.BlockSpec((NC, _TCR, PW), lambda i: (0, i, 0)),
          pl.BlockSpec((NC, _TCR, CW), lambda i: (0, i, 0)),
          pl.BlockSpec((_TCR, PW), lambda i: (i, 0)),
          pl.BlockSpec((1, PW), lambda i: (0, 0)),
      ],
      out_specs=pl.BlockSpec((_TCR, PW), lambda i: (i, 0)),
      out_shape=jax.ShapeDtypeStruct((N, PW), FP32),
  )(sump, cnt, q, b2)


def kernel(x, edge_index, W1l, b1l, W1r, W2l, b2l, W2r):
  e2 = edge_index.astype(jnp.int32).reshape(2, NW, EPW)
  eim = e2.reshape(2, NW, NCHUNK, CH).transpose(1, 2, 0, 3)
  zrows_d = jnp.zeros((RPT, D), FP32)
  zrows_p = jnp.zeros((RPT, PW), FP32)
  zcnt = jnp.zeros((RPT, CW), FP32)
  ones_h = jnp.ones((CH, CW), FP32)
  sums, cnts = _agg_l1(x, eim, zrows_d, zcnt, ones_h)
  w1lt = W1l.T
  w1rt = W1r.T
  w2lt = jnp.zeros((D, PW), FP32).at[:, :2].set(W2l.T)
  w2rt = jnp.zeros((D, PW), FP32).at[:, :2].set(W2r.T)
  b1 = b1l.reshape(1, D)
  b2 = jnp.zeros((1, PW), FP32).at[0, :2].set(b2l)
  p, q = _tc1(sums, cnts, x, w1lt, b1, w1rt, w2lt, w2rt)
  sump = _agg_l2(p, eim, zrows_p)
  outp = _tc2(sump, cnts, q, b2)
  return outp[:, :2]


# speedup vs baseline: 1.2526x; 1.0963x over previous
import jax
import jax.numpy as jnp
from jax import lax
from jax.experimental import pallas as pl
from jax.experimental.pallas import tpu as pltpu
from jax.experimental.pallas import tpu_sc as plsc

N = 10000
NP = 10240
E = 320000
D = 128
PW = 16
NC, NS = 2, 16
NW = NC * NS
EPW = E // NW
CH = 125
NCHUNK = EPW // CH
CW = 8
RPT = NP // NS
FP32 = jnp.float32


def _make_agg(width, with_counts):
  mesh = plsc.VectorSubcoreMesh(
      core_axis_name="c", subcore_axis_name="s", num_cores=NC, num_subcores=NS)
  scratch = [
      pltpu.VMEM((CH, width), FP32),
      pltpu.VMEM((CH, width), FP32),
      pltpu.VMEM((2, CH), jnp.int32),
      pltpu.VMEM((2, CH), jnp.int32),
      pltpu.VMEM((2, CH), jnp.int32),
      pltpu.VMEM((2, CH), jnp.int32),
      pltpu.VMEM_SHARED((NP, width), FP32),
  ] + [pltpu.SemaphoreType.DMA] * 8
  out_types = [jax.ShapeDtypeStruct((NC, NP, width), FP32)]
  if with_counts:
    scratch += [
        pltpu.VMEM((CH, CW), FP32),
        pltpu.VMEM_SHARED((NP, CW), FP32),
    ]
    out_types.append(jax.ShapeDtypeStruct((NC, NP, CW), FP32))

  def body(table, eim, zrows, *refs):
    if with_counts:
      (zcnt, ones_h, sum_out, cnt_out, rows0, rows1, is0, is1, is2, is3,
       acc_sh, i0, i1, i2, i3, g0, g1, ss0, ss1, ones_v, cnt_sh) = refs
    else:
      (sum_out, rows0, rows1, is0, is1, is2, is3,
       acc_sh, i0, i1, i2, i3, g0, g1, ss0, ss1) = refs
      zcnt = ones_h = cnt_out = ones_v = cnt_sh = None
    rows = (rows0, rows1)
    islot = (is0, is1, is2, is3)
    isem = (i0, i1, i2, i3)
    gsem = (g0, g1)
    ssem = (ss0, ss1)
    c = lax.axis_index("c")
    s = lax.axis_index("s")
    wid = s * NC + c

    base = s * RPT
    pltpu.sync_copy(zrows, acc_sh.at[pl.ds(base, RPT)])
    if with_counts:
      pltpu.sync_copy(ones_h, ones_v)
      pltpu.sync_copy(zcnt, cnt_sh.at[pl.ds(base, RPT)])
    plsc.subcore_barrier()

    for cc in (0, 1):
      pltpu.async_copy(eim.at[wid, cc], islot[cc], isem[cc])

    def do_chunk(cdyn, k):
      b = k % 2
      s2 = (k + 2) % 4
      pltpu.make_async_copy(eim.at[wid, 0], islot[k], isem[k]).wait()
      @pl.when(cdyn >= 2)
      def _():
        pltpu.make_async_copy(rows[b], acc_sh.at[pl.ds(0, CH)], ssem[b]).wait()
        if with_counts:
          pltpu.make_async_copy(
              ones_v, cnt_sh.at[pl.ds(0, CH)], ssem[b]).wait()
      gd = pltpu.async_copy(table.at[islot[k].at[0]], rows[b], gsem[b])
      @pl.when(cdyn + 2 < NCHUNK)
      def _():
        pltpu.async_copy(eim.at[wid, cdyn + 2], islot[s2], isem[s2])
      gd.wait()
      pltpu.async_copy(rows[b], acc_sh.at[islot[k].at[1]], ssem[b], add=True)
      if with_counts:
        pltpu.async_copy(ones_v, cnt_sh.at[islot[k].at[1]], ssem[b], add=True)

    def quad(t, _):
      cb = t * 4
      for k in range(4):
        do_chunk(cb + k, k)
      return 0
    lax.fori_loop(0, NCHUNK // 4, quad, 0)
    for b in (0, 1):
      pltpu.make_async_copy(rows[b], acc_sh.at[pl.ds(0, CH)], ssem[b]).wait()
      if with_counts:
        pltpu.make_async_copy(ones_v, cnt_sh.at[pl.ds(0, CH)], ssem[b]).wait()
    plsc.subcore_barrier()

    pltpu.sync_copy(acc_sh.at[pl.ds(base, RPT)], sum_out.at[c, pl.ds(base, RPT)])
    if with_counts:
      pltpu.sync_copy(cnt_sh.at[pl.ds(base, RPT)],
                      cnt_out.at[c, pl.ds(base, RPT)])

  out_type = tuple(out_types) if with_counts else out_types[0]
  return pl.kernel(
      body, out_type=out_type, mesh=mesh, scratch_types=scratch,
      compiler_params=pltpu.CompilerParams(use_tc_tiling_on_sc=False))


_agg_l1 = _make_agg(D, with_counts=True)
_agg_l2 = _make_agg(PW, with_counts=False)

_TCR = 1000


def _tc1_body(acc_ref, cnt_ref, x_ref, w1l_ref, b1_ref, w1r_ref,
              w2l_ref, w2r_ref, p_ref, q_ref):
  cnt = jnp.maximum(cnt_ref[0][:, 0:1] + cnt_ref[1][:, 0:1], 1.0)
  mean = (acc_ref[0] + acc_ref[1]) / cnt
  h = (jnp.dot(mean, w1l_ref[...], preferred_element_type=FP32)
       + b1_ref[...]
       + jnp.dot(x_ref[...], w1r_ref[...], preferred_element_type=FP32))
  h = jnp.maximum(h, 0.0)
  p_ref[...] = jnp.dot(h, w2l_ref[...], preferred_element_type=FP32)
  q_ref[...] = jnp.dot(h, w2r_ref[...], preferred_element_type=FP32)


def _tc1(acc, cnt, x, w1lt, b1, w1rt, w2lt, w2rt):
  grid = (N // _TCR,)
  return pl.pallas_call(
      _tc1_body,
      grid=grid,
      in_specs=[
          pl.BlockSpec((NC, _TCR, D), lambda i: (0, i, 0)),
          pl.BlockSpec((NC, _TCR, CW), lambda i: (0, i, 0)),
          pl.BlockSpec((_TCR, D), lambda i: (i, 0)),
          pl.BlockSpec((D, D), lambda i: (0, 0)),
          pl.BlockSpec((1, D), lambda i: (0, 0)),
          pl.BlockSpec((D, D), lambda i: (0, 0)),
          pl.BlockSpec((D, PW), lambda i: (0, 0)),
          pl.BlockSpec((D, PW), lambda i: (0, 0)),
      ],
      out_specs=[
          pl.BlockSpec((_TCR, PW), lambda i: (i, 0)),
          pl.BlockSpec((_TCR, PW), lambda i: (i, 0)),
      ],
      out_shape=[
          jax.ShapeDtypeStruct((N, PW), FP32),
          jax.ShapeDtypeStruct((N, PW), FP32),
      ],
  )(acc, cnt, x, w1lt, b1, w1rt, w2lt, w2rt)


def _tc2_body(sump_ref, cnt_ref, q_ref, b2_ref, out_ref):
  cnt = jnp.maximum(cnt_ref[0][:, 0:1] + cnt_ref[1][:, 0:1], 1.0)
  t = (sump_ref[0] + sump_ref[1]) / cnt + q_ref[...] + b2_ref[...]
  col = lax.broadcasted_iota(jnp.int32, t.shape, 1)
  valid = col < 2
  tm = jnp.where(valid, t, -jnp.inf)
  m = jnp.max(tm, axis=1, keepdims=True)
  ssum = jnp.sum(jnp.where(valid, jnp.exp(t - m), 0.0), axis=1, keepdims=True)
  out_ref[...] = t - m - jnp.log(ssum)


def _tc2(sump, cnt, q, b2):
  grid = (N // _TCR,)
  return pl.pallas_call(
      _tc2_body,
      grid=grid,
      in_specs=[
          pl.BlockSpec((NC, _TCR, PW), lambda i: (0, i, 0)),
          pl.BlockSpec((NC, _TCR, CW), lambda i: (0, i, 0)),
          pl.BlockSpec((_TCR, PW), lambda i: (i, 0)),
          pl.BlockSpec((1, PW), lambda i: (0, 0)),
      ],
      out_specs=pl.BlockSpec((_TCR, PW), lambda i: (i, 0)),
      out_shape=jax.ShapeDtypeStruct((N, PW), FP32),
  )(sump, cnt, q, b2)


def kernel(x, edge_index, W1l, b1l, W1r, W2l, b2l, W2r):
  e2 = edge_index.astype(jnp.int32).reshape(2, NW, EPW)
  eim = e2.reshape(2, NW, NCHUNK, CH).transpose(1, 2, 0, 3)
  zrows_d = jnp.zeros((RPT, D), FP32)
  zrows_p = jnp.zeros((RPT, PW), FP32)
  zcnt = jnp.zeros((RPT, CW), FP32)
  ones_h = jnp.ones((CH, CW), FP32)
  sums, cnts = _agg_l1(x, eim, zrows_d, zcnt, ones_h)
  w1lt = W1l.T
  w1rt = W1r.T
  w2lt = jnp.zeros((D, PW), FP32).at[:, :2].set(W2l.T)
  w2rt = jnp.zeros((D, PW), FP32).at[:, :2].set(W2r.T)
  b1 = b1l.reshape(1, D)
  b2 = jnp.zeros((1, PW), FP32).at[0, :2].set(b2l)
  p, q = _tc1(sums, cnts, x, w1lt, b1, w1rt, w2lt, w2rt)
  sump = _agg_l2(p, eim, zrows_p)
  outp = _tc2(sump, cnts, q, b2)
  return outp[:, :2]
